# baseline (speedup 1.0000x reference)
import functools

import jax
import jax.numpy as jnp
from jax import lax
from jax.experimental import pallas as pl
from jax.experimental.pallas import tpu as pltpu
from jax.experimental.pallas import tpu_sc as plsc

_NC = 2
_NS = 16
_NPAD = 10240
_HP = 32
_EPAD = 163840
_EPT = _EPAD // (_NC * _NS)
_CHUNK = 128
_GRP = 8
_GEDGES = _GRP * _CHUNK
_NGRP = _EPT // _GEDGES
_ZR = 128
_RPT = _NPAD // _NS
_MBLK = 1000


def _proj_body(x_ref, wl_ref, wr_ref, b_ref, y_ref, r_ref):
    xb = x_ref[...]
    y_ref[...] = jnp.dot(xb, wl_ref[...], preferred_element_type=jnp.float32,
                         precision=jax.lax.Precision.HIGHEST)
    r_ref[...] = jnp.dot(xb, wr_ref[...], preferred_element_type=jnp.float32,
                         precision=jax.lax.Precision.HIGHEST) + b_ref[...]


def _mid_body(p_ref, r_ref, wl2_ref, h_ref, y2_ref):
    h = jnp.maximum(p_ref[0] + p_ref[1] + r_ref[...], 0.0)
    h_ref[...] = h
    y2_ref[...] = jnp.dot(h, wl2_ref[...], preferred_element_type=jnp.float32,
                          precision=jax.lax.Precision.HIGHEST)


def _out_body(p_ref, h_ref, wr2_ref, b2_ref, w3_ref, b3_ref, o_ref):
    h2 = jnp.maximum(
        p_ref[0] + p_ref[1]
        + jnp.dot(h_ref[...], wr2_ref[...], preferred_element_type=jnp.float32,
                  precision=jax.lax.Precision.HIGHEST)
        + b2_ref[...],
        0.0,
    )
    o_ref[...] = jnp.dot(h2, w3_ref[...], preferred_element_type=jnp.float32,
                         precision=jax.lax.Precision.HIGHEST) + b3_ref[...]


def _sc_scatter(y, src, dst, wts):
    mesh = plsc.VectorSubcoreMesh(core_axis_name="c", subcore_axis_name="s")

    @functools.partial(
        pl.kernel,
        mesh=mesh,
        out_type=jax.ShapeDtypeStruct((_NC * _NPAD, _HP), jnp.float32),
        compiler_params=pltpu.CompilerParams(use_tc_tiling_on_sc=False),
        scratch_types=[
            pltpu.VMEM((_EPT,), jnp.int32),
            pltpu.VMEM((_EPT // _CHUNK, _CHUNK), jnp.int32),
            pltpu.VMEM((_EPT,), jnp.float32),
            pltpu.VMEM((_GEDGES, _HP), jnp.float32),
            pltpu.VMEM((_GEDGES, _HP), jnp.float32),
            pltpu.VMEM((_RPT, _HP), jnp.float32),
            pltpu.VMEM_SHARED((_NPAD, _HP), jnp.float32),
            pltpu.SemaphoreType.DMA,
            pltpu.SemaphoreType.DMA,
        ],
    )
    def run(y_hbm, src_hbm, dst_hbm, w_hbm, out_hbm,
            src_v, dst_v, w_v, rows_a, rows_b, zb_v, acc_sh, gsem, ssem):
        c = lax.axis_index("c")
        s = lax.axis_index("s")
        tid = c * _NS + s
        nchunks = _EPT // _CHUNK

        pltpu.async_copy(src_hbm.at[pl.ds(tid * _EPT, _EPT)], src_v, gsem)
        pltpu.async_copy(dst_hbm.at[pl.ds(tid * nchunks, nchunks)], dst_v, gsem)
        pltpu.async_copy(w_hbm.at[pl.ds(tid * _EPT, _EPT)], w_v, gsem)

        z16 = jnp.zeros((16,), jnp.float32)
        for i in range(_ZR):
            zb_v[i, 0:16] = z16
            zb_v[i, 16:32] = z16

        pltpu.make_async_copy(src_hbm.at[pl.ds(tid * _EPT, _EPT)], src_v, gsem).wait()
        pltpu.make_async_copy(
            dst_hbm.at[pl.ds(tid * nchunks, nchunks)], dst_v, gsem
        ).wait()
        pltpu.make_async_copy(w_hbm.at[pl.ds(tid * _EPT, _EPT)], w_v, gsem).wait()

        for j in range(_RPT // _ZR):
            pltpu.async_copy(
                zb_v.at[pl.ds(0, _ZR)],
                acc_sh.at[pl.ds(s * _RPT + j * _ZR, _ZR)],
                ssem,
            )
        for j in range(_RPT // _ZR):
            pltpu.make_async_copy(
                zb_v.at[pl.ds(0, _ZR)],
                acc_sh.at[pl.ds(s * _RPT + j * _ZR, _ZR)],
                ssem,
            ).wait()
        plsc.subcore_barrier()

        def fire_gathers(g, buf):
            for b in range(_GRP):
                pltpu.async_copy(
                    y_hbm.at[src_v.at[pl.ds((g * _GRP + b) * _CHUNK, _CHUNK)]],
                    buf.at[pl.ds(b * _CHUNK, _CHUNK)],
                    gsem,
                )

        def drain_gathers(g, buf):
            for b in range(_GRP):
                pltpu.make_async_copy(
                    y_hbm.at[src_v.at[pl.ds((g * _GRP + b) * _CHUNK, _CHUNK)]],
                    buf.at[pl.ds(b * _CHUNK, _CHUNK)],
                    gsem,
                ).wait()

        def scale_and_scatter(g, buf):
            @pl.loop(0, _GEDGES // 16)
            def scale(q):
                wv = w_v[pl.ds(g * _GEDGES + q * 16, 16)]
                for jj in range(16):
                    w = wv[jj]
                    buf[q * 16 + jj, 0:16] = buf[q * 16 + jj, 0:16] * w
                    buf[q * 16 + jj, 16:32] = buf[q * 16 + jj, 16:32] * w

            for b in range(_GRP):
                pltpu.async_copy(
                    buf.at[pl.ds(b * _CHUNK, _CHUNK)],
                    acc_sh.at[dst_v.at[g * _GRP + b]],
                    ssem,
                    add=True,
                )
            for b in range(_GRP):
                pltpu.make_async_copy(
                    buf.at[pl.ds(b * _CHUNK, _CHUNK)],
                    acc_sh.at[dst_v.at[g * _GRP + b]],
                    ssem,
                ).wait()

        fire_gathers(0, rows_a)

        @pl.loop(0, (_NGRP - 1) // 2)
        def pipelined(j2):
            g0 = j2 * 2
            drain_gathers(g0, rows_a)
            fire_gathers(g0 + 1, rows_b)
            scale_and_scatter(g0, rows_a)
            drain_gathers(g0 + 1, rows_b)
            fire_gathers(g0 + 2, rows_a)
            scale_and_scatter(g0 + 1, rows_b)

        glast = _NGRP - 1
        drain_gathers(glast, rows_a)
        scale_and_scatter(glast, rows_a)

        plsc.subcore_barrier()

        pltpu.async_copy(acc_sh.at[pl.ds(s * _RPT, _RPT)], zb_v, gsem).wait()
        pltpu.async_copy(
            zb_v, out_hbm.at[pl.ds(c * _NPAD + s * _RPT, _RPT)], gsem
        ).wait()

    return run(y, src, dst, wts).reshape(_NC, _NPAD, _HP)


def kernel(x, edge_index, edge_weights, Wl1, bl1, Wr1, br1, Wl2, bl2, Wr2, br2, W3, b3):
    N, D = x.shape
    H = Wl1.shape[1]
    C = W3.shape[1]
    E = edge_index.shape[1]

    src = jnp.zeros((_EPAD,), jnp.int32).at[:E].set(edge_index[0])
    dst = jnp.zeros((_EPAD,), jnp.int32).at[:E].set(edge_index[1]).reshape(
        _EPAD // _CHUNK, _CHUNK)
    wts = jnp.zeros((_EPAD,), jnp.float32).at[:E].set(edge_weights)

    Wl1p = jnp.zeros((D, _HP), jnp.float32).at[:, :H].set(Wl1)
    Wr1p = jnp.zeros((D, _HP), jnp.float32).at[:, :H].set(Wr1)
    b1p = jnp.zeros((1, _HP), jnp.float32).at[0, :H].set(bl1 + br1)
    Wl2p = jnp.zeros((_HP, _HP), jnp.float32).at[:H, :H].set(Wl2)
    Wr2p = jnp.zeros((_HP, _HP), jnp.float32).at[:H, :H].set(Wr2)
    b2p = jnp.zeros((1, _HP), jnp.float32).at[0, :H].set(bl2 + br2)
    W3p = jnp.zeros((_HP, C), jnp.float32).at[:H].set(W3)
    b3p = b3[None, :]

    grid = N // _MBLK

    y1, r1 = pl.pallas_call(
        _proj_body,
        grid=(grid,),
        in_specs=[
            pl.BlockSpec((_MBLK, D), lambda i: (i, 0)),
            pl.BlockSpec((D, _HP), lambda i: (0, 0)),
            pl.BlockSpec((D, _HP), lambda i: (0, 0)),
            pl.BlockSpec((1, _HP), lambda i: (0, 0)),
        ],
        out_specs=[
            pl.BlockSpec((_MBLK, _HP), lambda i: (i, 0)),
            pl.BlockSpec((_MBLK, _HP), lambda i: (i, 0)),
        ],
        out_shape=[
            jax.ShapeDtypeStruct((N, _HP), jnp.float32),
            jax.ShapeDtypeStruct((N, _HP), jnp.float32),
        ],
    )(x, Wl1p, Wr1p, b1p)

    part1 = _sc_scatter(y1, src, dst, wts)

    h1, y2 = pl.pallas_call(
        _mid_body,
        grid=(grid,),
        in_specs=[
            pl.BlockSpec((_NC, _MBLK, _HP), lambda i: (0, i, 0)),
            pl.BlockSpec((_MBLK, _HP), lambda i: (i, 0)),
            pl.BlockSpec((_HP, _HP), lambda i: (0, 0)),
        ],
        out_specs=[
            pl.BlockSpec((_MBLK, _HP), lambda i: (i, 0)),
            pl.BlockSpec((_MBLK, _HP), lambda i: (i, 0)),
        ],
        out_shape=[
            jax.ShapeDtypeStruct((N, _HP), jnp.float32),
            jax.ShapeDtypeStruct((N, _HP), jnp.float32),
        ],
    )(part1, r1, Wl2p)

    part2 = _sc_scatter(y2, src, dst, wts)

    out_p = pl.pallas_call(
        _out_body,
        grid=(grid,),
        in_specs=[
            pl.BlockSpec((_NC, _MBLK, _HP), lambda i: (0, i, 0)),
            pl.BlockSpec((_MBLK, _HP), lambda i: (i, 0)),
            pl.BlockSpec((_HP, _HP), lambda i: (0, 0)),
            pl.BlockSpec((1, _HP), lambda i: (0, 0)),
            pl.BlockSpec((_HP, C), lambda i: (0, 0)),
            pl.BlockSpec((1, C), lambda i: (0, 0)),
        ],
        out_specs=pl.BlockSpec((_MBLK, C), lambda i: (i, 0)),
        out_shape=jax.ShapeDtypeStruct((N, C), jnp.float32),
    )(part2, h1, Wr2p, b2p, W3p, b3p)

    return out_p


# speedup vs baseline: 1.1369x; 1.1369x over previous
import functools

import jax
import jax.numpy as jnp
from jax import lax
from jax.experimental import pallas as pl
from jax.experimental.pallas import tpu as pltpu
from jax.experimental.pallas import tpu_sc as plsc

_NC = 2
_NS = 16
_NPAD = 10240
_HP = 32
_EPAD = 163840
_EPT = _EPAD // (_NC * _NS)
_CHUNK = 128
_GRP = 8
_GEDGES = _GRP * _CHUNK
_NGRP = _EPT // _GEDGES
_ZR = 128
_RPT = _NPAD // _NS
_MBLK = 1000


def _proj_body(x_ref, wl_ref, wr_ref, b_ref, y_ref, r_ref):
    xb = x_ref[...]
    y_ref[...] = jnp.dot(xb, wl_ref[...], preferred_element_type=jnp.float32)
    r_ref[...] = jnp.dot(xb, wr_ref[...], preferred_element_type=jnp.float32) + b_ref[...]


def _mid_body(p_ref, r_ref, wl2_ref, h_ref, y2_ref):
    h = jnp.maximum(p_ref[0] + p_ref[1] + r_ref[...], 0.0)
    h_ref[...] = h
    y2_ref[...] = jnp.dot(h, wl2_ref[...], preferred_element_type=jnp.float32)


def _out_body(p_ref, h_ref, wr2_ref, b2_ref, w3_ref, b3_ref, o_ref):
    h2 = jnp.maximum(
        p_ref[0] + p_ref[1]
        + jnp.dot(h_ref[...], wr2_ref[...], preferred_element_type=jnp.float32)
        + b2_ref[...],
        0.0,
    )
    o_ref[...] = jnp.dot(h2, w3_ref[...], preferred_element_type=jnp.float32) + b3_ref[...]


def _sc_scatter(y, src, dst, wts):
    mesh = plsc.VectorSubcoreMesh(core_axis_name="c", subcore_axis_name="s")

    @functools.partial(
        pl.kernel,
        mesh=mesh,
        out_type=jax.ShapeDtypeStruct((_NC * _NPAD, _HP), jnp.float32),
        compiler_params=pltpu.CompilerParams(use_tc_tiling_on_sc=False),
        scratch_types=[
            pltpu.VMEM((_EPT,), jnp.int32),
            pltpu.VMEM((_EPT // _CHUNK, _CHUNK), jnp.int32),
            pltpu.VMEM((_EPT,), jnp.float32),
            pltpu.VMEM((_GEDGES, _HP), jnp.float32),
            pltpu.VMEM((_GEDGES, _HP), jnp.float32),
            pltpu.VMEM((_RPT, _HP), jnp.float32),
            pltpu.VMEM_SHARED((_NPAD, _HP), jnp.float32),
            pltpu.SemaphoreType.DMA,
            pltpu.SemaphoreType.DMA,
        ],
    )
    def run(y_hbm, src_hbm, dst_hbm, w_hbm, out_hbm,
            src_v, dst_v, w_v, rows_a, rows_b, zb_v, acc_sh, gsem, ssem):
        c = lax.axis_index("c")
        s = lax.axis_index("s")
        tid = c * _NS + s
        nchunks = _EPT // _CHUNK

        pltpu.async_copy(src_hbm.at[pl.ds(tid * _EPT, _EPT)], src_v, gsem)
        pltpu.async_copy(dst_hbm.at[pl.ds(tid * nchunks, nchunks)], dst_v, gsem)
        pltpu.async_copy(w_hbm.at[pl.ds(tid * _EPT, _EPT)], w_v, gsem)

        z16 = jnp.zeros((16,), jnp.float32)
        for i in range(_ZR):
            zb_v[i, 0:16] = z16
            zb_v[i, 16:32] = z16

        pltpu.make_async_copy(src_hbm.at[pl.ds(tid * _EPT, _EPT)], src_v, gsem).wait()
        pltpu.make_async_copy(
            dst_hbm.at[pl.ds(tid * nchunks, nchunks)], dst_v, gsem
        ).wait()
        pltpu.make_async_copy(w_hbm.at[pl.ds(tid * _EPT, _EPT)], w_v, gsem).wait()

        for j in range(_RPT // _ZR):
            pltpu.async_copy(
                zb_v.at[pl.ds(0, _ZR)],
                acc_sh.at[pl.ds(s * _RPT + j * _ZR, _ZR)],
                ssem,
            )
        for j in range(_RPT // _ZR):
            pltpu.make_async_copy(
                zb_v.at[pl.ds(0, _ZR)],
                acc_sh.at[pl.ds(s * _RPT + j * _ZR, _ZR)],
                ssem,
            ).wait()
        plsc.subcore_barrier()

        def fire_gathers(g, buf):
            for b in range(_GRP):
                pltpu.async_copy(
                    y_hbm.at[src_v.at[pl.ds((g * _GRP + b) * _CHUNK, _CHUNK)]],
                    buf.at[pl.ds(b * _CHUNK, _CHUNK)],
                    gsem,
                )

        def drain_gathers(g, buf):
            for b in range(_GRP):
                pltpu.make_async_copy(
                    y_hbm.at[src_v.at[pl.ds((g * _GRP + b) * _CHUNK, _CHUNK)]],
                    buf.at[pl.ds(b * _CHUNK, _CHUNK)],
                    gsem,
                ).wait()

        def scale_and_scatter(g, buf):
            @pl.loop(0, _GEDGES // 16)
            def scale(q):
                wv = w_v[pl.ds(g * _GEDGES + q * 16, 16)]
                for jj in range(16):
                    w = wv[jj]
                    buf[q * 16 + jj, 0:16] = buf[q * 16 + jj, 0:16] * w
                    buf[q * 16 + jj, 16:32] = buf[q * 16 + jj, 16:32] * w

            for b in range(_GRP):
                pltpu.async_copy(
                    buf.at[pl.ds(b * _CHUNK, _CHUNK)],
                    acc_sh.at[dst_v.at[g * _GRP + b]],
                    ssem,
                    add=True,
                )
            for b in range(_GRP):
                pltpu.make_async_copy(
                    buf.at[pl.ds(b * _CHUNK, _CHUNK)],
                    acc_sh.at[dst_v.at[g * _GRP + b]],
                    ssem,
                ).wait()

        fire_gathers(0, rows_a)

        @pl.loop(0, (_NGRP - 1) // 2)
        def pipelined(j2):
            g0 = j2 * 2
            drain_gathers(g0, rows_a)
            fire_gathers(g0 + 1, rows_b)
            scale_and_scatter(g0, rows_a)
            drain_gathers(g0 + 1, rows_b)
            fire_gathers(g0 + 2, rows_a)
            scale_and_scatter(g0 + 1, rows_b)

        glast = _NGRP - 1
        drain_gathers(glast, rows_a)
        scale_and_scatter(glast, rows_a)

        plsc.subcore_barrier()

        pltpu.async_copy(acc_sh.at[pl.ds(s * _RPT, _RPT)], zb_v, gsem).wait()
        pltpu.async_copy(
            zb_v, out_hbm.at[pl.ds(c * _NPAD + s * _RPT, _RPT)], gsem
        ).wait()

    return run(y, src, dst, wts).reshape(_NC, _NPAD, _HP)


def kernel(x, edge_index, edge_weights, Wl1, bl1, Wr1, br1, Wl2, bl2, Wr2, br2, W3, b3):
    N, D = x.shape
    H = Wl1.shape[1]
    C = W3.shape[1]
    E = edge_index.shape[1]

    src = jnp.zeros((_EPAD,), jnp.int32).at[:E].set(edge_index[0])
    dst = jnp.zeros((_EPAD,), jnp.int32).at[:E].set(edge_index[1]).reshape(
        _EPAD // _CHUNK, _CHUNK)
    wts = jnp.zeros((_EPAD,), jnp.float32).at[:E].set(edge_weights)

    Wl1p = jnp.zeros((D, _HP), jnp.float32).at[:, :H].set(Wl1)
    Wr1p = jnp.zeros((D, _HP), jnp.float32).at[:, :H].set(Wr1)
    b1p = jnp.zeros((1, _HP), jnp.float32).at[0, :H].set(bl1 + br1)
    Wl2p = jnp.zeros((_HP, _HP), jnp.float32).at[:H, :H].set(Wl2)
    Wr2p = jnp.zeros((_HP, _HP), jnp.float32).at[:H, :H].set(Wr2)
    b2p = jnp.zeros((1, _HP), jnp.float32).at[0, :H].set(bl2 + br2)
    W3p = jnp.zeros((_HP, C), jnp.float32).at[:H].set(W3)
    b3p = b3[None, :]

    grid = N // _MBLK

    y1, r1 = pl.pallas_call(
        _proj_body,
        grid=(grid,),
        in_specs=[
            pl.BlockSpec((_MBLK, D), lambda i: (i, 0)),
            pl.BlockSpec((D, _HP), lambda i: (0, 0)),
            pl.BlockSpec((D, _HP), lambda i: (0, 0)),
            pl.BlockSpec((1, _HP), lambda i: (0, 0)),
        ],
        out_specs=[
            pl.BlockSpec((_MBLK, _HP), lambda i: (i, 0)),
            pl.BlockSpec((_MBLK, _HP), lambda i: (i, 0)),
        ],
        out_shape=[
            jax.ShapeDtypeStruct((N, _HP), jnp.float32),
            jax.ShapeDtypeStruct((N, _HP), jnp.float32),
        ],
    )(x, Wl1p, Wr1p, b1p)

    part1 = _sc_scatter(y1, src, dst, wts)

    h1, y2 = pl.pallas_call(
        _mid_body,
        grid=(grid,),
        in_specs=[
            pl.BlockSpec((_NC, _MBLK, _HP), lambda i: (0, i, 0)),
            pl.BlockSpec((_MBLK, _HP), lambda i: (i, 0)),
            pl.BlockSpec((_HP, _HP), lambda i: (0, 0)),
        ],
        out_specs=[
            pl.BlockSpec((_MBLK, _HP), lambda i: (i, 0)),
            pl.BlockSpec((_MBLK, _HP), lambda i: (i, 0)),
        ],
        out_shape=[
            jax.ShapeDtypeStruct((N, _HP), jnp.float32),
            jax.ShapeDtypeStruct((N, _HP), jnp.float32),
        ],
    )(part1, r1, Wl2p)

    part2 = _sc_scatter(y2, src, dst, wts)

    out_p = pl.pallas_call(
        _out_body,
        grid=(grid,),
        in_specs=[
            pl.BlockSpec((_NC, _MBLK, _HP), lambda i: (0, i, 0)),
            pl.BlockSpec((_MBLK, _HP), lambda i: (i, 0)),
            pl.BlockSpec((_HP, _HP), lambda i: (0, 0)),
            pl.BlockSpec((1, _HP), lambda i: (0, 0)),
            pl.BlockSpec((_HP, C), lambda i: (0, 0)),
            pl.BlockSpec((1, C), lambda i: (0, 0)),
        ],
        out_specs=pl.BlockSpec((_MBLK, C), lambda i: (i, 0)),
        out_shape=jax.ShapeDtypeStruct((N, C), jnp.float32),
    )(part2, h1, Wr2p, b2p, W3p, b3p)

    return out_p


# speedup vs baseline: 1.4235x; 1.2520x over previous
import functools

import jax
import jax.numpy as jnp
from jax import lax
from jax.experimental import pallas as pl
from jax.experimental.pallas import tpu as pltpu
from jax.experimental.pallas import tpu_sc as plsc

_NC = 2
_NS = 16
_NPAD = 10240
_HP = 32
_EPAD = 163840
_EPT = _EPAD // (_NC * _NS)
_CHUNK = 128
_GRP = 8
_GEDGES = _GRP * _CHUNK
_NGRP = _EPT // _GEDGES
_ZR = 128
_RPT = _NPAD // _NS
_MBLK = 1000


def _proj_body(x_ref, wl_ref, wr_ref, b_ref, y_ref, r_ref):
    xb = x_ref[...]
    y_ref[...] = jnp.dot(xb, wl_ref[...], preferred_element_type=jnp.float32).astype(jnp.bfloat16)
    r_ref[...] = jnp.dot(xb, wr_ref[...], preferred_element_type=jnp.float32) + b_ref[...]


def _mid_body(p_ref, r_ref, wl2_ref, h_ref, y2_ref):
    h = jnp.maximum(p_ref[0] + p_ref[1] + r_ref[...], 0.0)
    h_ref[...] = h
    y2_ref[...] = jnp.dot(h, wl2_ref[...], preferred_element_type=jnp.float32).astype(jnp.bfloat16)


def _out_body(p_ref, h_ref, wr2_ref, b2_ref, w3_ref, b3_ref, o_ref):
    h2 = jnp.maximum(
        p_ref[0] + p_ref[1]
        + jnp.dot(h_ref[...], wr2_ref[...], preferred_element_type=jnp.float32)
        + b2_ref[...],
        0.0,
    )
    o_ref[...] = jnp.dot(h2, w3_ref[...], preferred_element_type=jnp.float32) + b3_ref[...]


def _sc_scatter(y, src, dst, wts):
    mesh = plsc.VectorSubcoreMesh(core_axis_name="c", subcore_axis_name="s")

    @functools.partial(
        pl.kernel,
        mesh=mesh,
        out_type=jax.ShapeDtypeStruct((_NC * _NPAD, _HP), jnp.float32),
        compiler_params=pltpu.CompilerParams(use_tc_tiling_on_sc=False, needs_layout_passes=False),
        scratch_types=[
            pltpu.VMEM((_EPT,), jnp.int32),
            pltpu.VMEM((_EPT // _CHUNK, _CHUNK), jnp.int32),
            pltpu.VMEM((_EPT,), jnp.float32),
            pltpu.VMEM((_GEDGES, _HP), jnp.bfloat16),
            pltpu.VMEM((_GEDGES, _HP), jnp.bfloat16),
            pltpu.VMEM((_GEDGES, _HP), jnp.float32),
            pltpu.VMEM((_RPT, _HP), jnp.float32),
            pltpu.VMEM_SHARED((_NPAD, _HP), jnp.float32),
            pltpu.SemaphoreType.DMA,
            pltpu.SemaphoreType.DMA,
        ],
    )
    def run(y_hbm, src_hbm, dst_hbm, w_hbm, out_hbm,
            src_v, dst_v, w_v, rows_a, rows_b, msg_v, zb_v, acc_sh, gsem, ssem):
        c = lax.axis_index("c")
        s = lax.axis_index("s")
        tid = c * _NS + s
        nchunks = _EPT // _CHUNK

        pltpu.async_copy(src_hbm.at[pl.ds(tid * _EPT, _EPT)], src_v, gsem)
        pltpu.async_copy(dst_hbm.at[pl.ds(tid * nchunks, nchunks)], dst_v, gsem)
        pltpu.async_copy(w_hbm.at[pl.ds(tid * _EPT, _EPT)], w_v, gsem)

        z16 = jnp.zeros((16,), jnp.float32)
        for i in range(_ZR):
            zb_v[i, 0:16] = z16
            zb_v[i, 16:32] = z16

        pltpu.make_async_copy(src_hbm.at[pl.ds(tid * _EPT, _EPT)], src_v, gsem).wait()
        pltpu.make_async_copy(
            dst_hbm.at[pl.ds(tid * nchunks, nchunks)], dst_v, gsem
        ).wait()
        pltpu.make_async_copy(w_hbm.at[pl.ds(tid * _EPT, _EPT)], w_v, gsem).wait()

        for j in range(_RPT // _ZR):
            pltpu.async_copy(
                zb_v.at[pl.ds(0, _ZR)],
                acc_sh.at[pl.ds(s * _RPT + j * _ZR, _ZR)],
                ssem,
            )
        for j in range(_RPT // _ZR):
            pltpu.make_async_copy(
                zb_v.at[pl.ds(0, _ZR)],
                acc_sh.at[pl.ds(s * _RPT + j * _ZR, _ZR)],
                ssem,
            ).wait()
        plsc.subcore_barrier()

        def fire_gathers(g, buf):
            for b in range(_GRP):
                pltpu.async_copy(
                    y_hbm.at[src_v.at[pl.ds((g * _GRP + b) * _CHUNK, _CHUNK)]],
                    buf.at[pl.ds(b * _CHUNK, _CHUNK)],
                    gsem,
                )

        def drain_gathers(g, buf):
            for b in range(_GRP):
                pltpu.make_async_copy(
                    y_hbm.at[src_v.at[pl.ds((g * _GRP + b) * _CHUNK, _CHUNK)]],
                    buf.at[pl.ds(b * _CHUNK, _CHUNK)],
                    gsem,
                ).wait()

        def scale_and_scatter(g, buf):
            @pl.loop(0, _GEDGES // 16)
            def scale(q):
                wv = w_v[pl.ds(g * _GEDGES + q * 16, 16)]
                for jj in range(16):
                    w = wv[jj]
                    row = buf[q * 16 + jj, 0:32]
                    lo, hi = plsc.unpack(row, format=plsc.PackFormat.INTERLEAVED)
                    msg_v[q * 16 + jj, 0:16] = lo * w
                    msg_v[q * 16 + jj, 16:32] = hi * w

            for b in range(_GRP):
                pltpu.async_copy(
                    msg_v.at[pl.ds(b * _CHUNK, _CHUNK)],
                    acc_sh.at[dst_v.at[g * _GRP + b]],
                    ssem,
                    add=True,
                )
            for b in range(_GRP):
                pltpu.make_async_copy(
                    msg_v.at[pl.ds(b * _CHUNK, _CHUNK)],
                    acc_sh.at[dst_v.at[g * _GRP + b]],
                    ssem,
                ).wait()

        fire_gathers(0, rows_a)

        @pl.loop(0, (_NGRP - 1) // 2)
        def pipelined(j2):
            g0 = j2 * 2
            drain_gathers(g0, rows_a)
            fire_gathers(g0 + 1, rows_b)
            scale_and_scatter(g0, rows_a)
            drain_gathers(g0 + 1, rows_b)
            fire_gathers(g0 + 2, rows_a)
            scale_and_scatter(g0 + 1, rows_b)

        glast = _NGRP - 1
        drain_gathers(glast, rows_a)
        scale_and_scatter(glast, rows_a)

        plsc.subcore_barrier()

        pltpu.async_copy(acc_sh.at[pl.ds(s * _RPT, _RPT)], zb_v, gsem).wait()
        pltpu.async_copy(
            zb_v, out_hbm.at[pl.ds(c * _NPAD + s * _RPT, _RPT)], gsem
        ).wait()

    return run(y, src, dst, wts).reshape(_NC, _NPAD, _HP)


def kernel(x, edge_index, edge_weights, Wl1, bl1, Wr1, br1, Wl2, bl2, Wr2, br2, W3, b3):
    N, D = x.shape
    H = Wl1.shape[1]
    C = W3.shape[1]
    E = edge_index.shape[1]

    src = jnp.zeros((_EPAD,), jnp.int32).at[:E].set(edge_index[0])
    dst = jnp.zeros((_EPAD,), jnp.int32).at[:E].set(edge_index[1]).reshape(
        _EPAD // _CHUNK, _CHUNK)
    wts = jnp.zeros((_EPAD,), jnp.float32).at[:E].set(edge_weights)

    perm = jnp.arange(_HP).reshape(2, _HP // 2).T.reshape(-1)
    Wl1p = jnp.zeros((D, _HP), jnp.float32).at[:, :H].set(Wl1)[:, perm]
    Wr1p = jnp.zeros((D, _HP), jnp.float32).at[:, :H].set(Wr1)
    b1p = jnp.zeros((1, _HP), jnp.float32).at[0, :H].set(bl1 + br1)
    Wl2p = jnp.zeros((_HP, _HP), jnp.float32).at[:H, :H].set(Wl2)[:, perm]
    Wr2p = jnp.zeros((_HP, _HP), jnp.float32).at[:H, :H].set(Wr2)
    b2p = jnp.zeros((1, _HP), jnp.float32).at[0, :H].set(bl2 + br2)
    W3p = jnp.zeros((_HP, C), jnp.float32).at[:H].set(W3)
    b3p = b3[None, :]

    grid = N // _MBLK

    y1, r1 = pl.pallas_call(
        _proj_body,
        grid=(grid,),
        in_specs=[
            pl.BlockSpec((_MBLK, D), lambda i: (i, 0)),
            pl.BlockSpec((D, _HP), lambda i: (0, 0)),
            pl.BlockSpec((D, _HP), lambda i: (0, 0)),
            pl.BlockSpec((1, _HP), lambda i: (0, 0)),
        ],
        out_specs=[
            pl.BlockSpec((_MBLK, _HP), lambda i: (i, 0)),
            pl.BlockSpec((_MBLK, _HP), lambda i: (i, 0)),
        ],
        out_shape=[
            jax.ShapeDtypeStruct((N, _HP), jnp.bfloat16),
            jax.ShapeDtypeStruct((N, _HP), jnp.float32),
        ],
    )(x, Wl1p, Wr1p, b1p)

    part1 = _sc_scatter(y1, src, dst, wts)

    h1, y2 = pl.pallas_call(
        _mid_body,
        grid=(grid,),
        in_specs=[
            pl.BlockSpec((_NC, _MBLK, _HP), lambda i: (0, i, 0)),
            pl.BlockSpec((_MBLK, _HP), lambda i: (i, 0)),
            pl.BlockSpec((_HP, _HP), lambda i: (0, 0)),
        ],
        out_specs=[
            pl.BlockSpec((_MBLK, _HP), lambda i: (i, 0)),
            pl.BlockSpec((_MBLK, _HP), lambda i: (i, 0)),
        ],
        out_shape=[
            jax.ShapeDtypeStruct((N, _HP), jnp.float32),
            jax.ShapeDtypeStruct((N, _HP), jnp.bfloat16),
        ],
    )(part1, r1, Wl2p)

    part2 = _sc_scatter(y2, src, dst, wts)

    out_p = pl.pallas_call(
        _out_body,
        grid=(grid,),
        in_specs=[
            pl.BlockSpec((_NC, _MBLK, _HP), lambda i: (0, i, 0)),
            pl.BlockSpec((_MBLK, _HP), lambda i: (i, 0)),
            pl.BlockSpec((_HP, _HP), lambda i: (0, 0)),
            pl.BlockSpec((1, _HP), lambda i: (0, 0)),
            pl.BlockSpec((_HP, C), lambda i: (0, 0)),
            pl.BlockSpec((1, C), lambda i: (0, 0)),
        ],
        out_specs=pl.BlockSpec((_MBLK, C), lambda i: (i, 0)),
        out_shape=jax.ShapeDtypeStruct((N, C), jnp.float32),
    )(part2, h1, Wr2p, b2p, W3p, b3p)

    return out_p
